# Initial kernel scaffold; baseline (speedup 1.0000x reference)
import jax
import jax.numpy as jnp
from jax.experimental import pallas as pl


def kernel(output, mask, ind, target):
    raise NotImplementedError("write your pallas kernel here")


# speedup vs baseline: 1.9945x; 1.9945x over previous
import functools

import jax
import jax.numpy as jnp
from jax import lax
from jax.experimental import pallas as pl
from jax.experimental.pallas import tpu as pltpu
from jax.experimental.pallas import tpu_sc as plsc

B = 16
C = 2
HW = 512 * 512
K = 500
KPAD = 512
KH = KPAD // 2
LANES = 16
NCHUNK = KH // LANES
ELEMS_PER_TILE = 2 * KH
GATHER_GROUP = 128
NGROUP = ELEMS_PER_TILE // GATHER_GROUP


def _sc_body(table_hbm, ind_hbm, maskf_hbm, tgt_hbm,
             part_out,
             ind_v, mask_v, tgt0_v, tgt1_v, idx_v, vals_v, acc_v, sem):
    nc = 2
    wid = lax.axis_index("s") * nc + lax.axis_index("c")
    b = wid // 2
    k0 = (wid % 2) * KH

    pltpu.sync_copy(ind_hbm.at[b, pl.ds(k0, KH)], ind_v)
    pltpu.sync_copy(maskf_hbm.at[b, pl.ds(k0, KH)], mask_v)
    pltpu.sync_copy(tgt_hbm.at[0, b, pl.ds(k0, KH)], tgt0_v)
    pltpu.sync_copy(tgt_hbm.at[1, b, pl.ds(k0, KH)], tgt1_v)

    base0 = b * (C * HW)
    for i in range(NCHUNK):
        ind_c = ind_v[pl.ds(i * LANES, LANES)]
        flat0 = base0 + ind_c
        g = i // 8
        o = (i % 8) * LANES
        idx_v[g, pl.ds(o, LANES)] = flat0
        idx_v[2 + g, pl.ds(o, LANES)] = flat0 + HW

    copies = [
        pltpu.async_copy(
            table_hbm.at[idx_v.at[j]],
            vals_v.at[pl.ds(j * GATHER_GROUP, GATHER_GROUP)],
            sem,
        )
        for j in range(NGROUP)
    ]
    for cp in copies:
        cp.wait()

    loss_acc = jnp.zeros((LANES,), jnp.float32)
    num_acc = jnp.zeros((LANES,), jnp.float32)
    for i in range(NCHUNK):
        sl = pl.ds(i * LANES, LANES)
        v0 = vals_v[sl]
        v1 = vals_v[pl.ds(KH + i * LANES, LANES)]
        m = mask_v[sl]
        d0 = (v0 - tgt0_v[sl]) * m
        d1 = (v1 - tgt1_v[sl]) * m
        a0 = jnp.abs(d0)
        a1 = jnp.abs(d1)
        e0 = jnp.where(a0 < 1.0, 0.5 * d0 * d0, a0 - 0.5)
        e1 = jnp.where(a1 < 1.0, 0.5 * d1 * d1, a1 - 0.5)
        loss_acc = loss_acc + e0 + e1
        num_acc = num_acc + m

    zero = jnp.zeros((LANES,), jnp.float32)
    acc_v[pl.ds(0, LANES)] = loss_acc
    acc_v[pl.ds(LANES, LANES)] = num_acc
    for z in range(2, 8):
        acc_v[pl.ds(z * LANES, LANES)] = zero
    pltpu.sync_copy(acc_v, part_out.at[wid])


@jax.jit
def _reg_loss_sc(table, ind_pad, maskf_pad, tgt_pad):
    mesh = plsc.VectorSubcoreMesh(core_axis_name="c", subcore_axis_name="s")
    k = functools.partial(
        pl.kernel,
        mesh=mesh,
        out_type=jax.ShapeDtypeStruct((32, 128), jnp.float32),
        scratch_types=[
            pltpu.VMEM((KH,), jnp.int32),
            pltpu.VMEM((KH,), jnp.float32),
            pltpu.VMEM((KH,), jnp.float32),
            pltpu.VMEM((KH,), jnp.float32),
            pltpu.VMEM((NGROUP, GATHER_GROUP), jnp.int32),
            pltpu.VMEM((ELEMS_PER_TILE,), jnp.float32),
            pltpu.VMEM((128,), jnp.float32),
            pltpu.SemaphoreType.DMA,
        ],
    )(_sc_body)
    part = k(table, ind_pad, maskf_pad, tgt_pad)
    num = part[:, LANES:2 * LANES].sum()
    return part[:, :LANES].sum() / (num + 0.0001)


def kernel(output, mask, ind, target):
    table = output.reshape(B * C * HW)
    ind_pad = jnp.pad(ind.astype(jnp.int32), ((0, 0), (0, KPAD - K)))
    maskf_pad = jnp.pad(mask.astype(jnp.float32), ((0, 0), (0, KPAD - K)))
    tgt_pad = jnp.pad(jnp.transpose(target, (2, 0, 1)),
                      ((0, 0), (0, 0), (0, KPAD - K)))
    return _reg_loss_sc(table, ind_pad, maskf_pad, tgt_pad)


# speedup vs baseline: 3.8872x; 1.9489x over previous
import functools

import jax
import jax.numpy as jnp
from jax import lax
from jax.experimental import pallas as pl
from jax.experimental.pallas import tpu as pltpu
from jax.experimental.pallas import tpu_sc as plsc

B = 16
C = 2
HW = 512 * 512
K = 500
KPAD = 512
KH = KPAD // 2
LANES = 16
NCHUNK = KH // LANES
ELEMS_PER_TILE = 2 * KH
GATHER_GROUP = 128
NGROUP = ELEMS_PER_TILE // GATHER_GROUP


def _sc_body(table_hbm, ind_hbm, maskf_hbm, tgt_hbm,
             part_out,
             ind_v, mask_v, tgt0_v, tgt1_v, idx_v, vals_v, acc_v, sem):
    nc = 2
    wid = lax.axis_index("s") * nc + lax.axis_index("c")
    b = wid // 2
    k0 = (wid % 2) * KH

    pltpu.sync_copy(ind_hbm.at[b, pl.ds(k0, KH)], ind_v)
    pltpu.sync_copy(maskf_hbm.at[b, pl.ds(k0, KH)], mask_v)
    pltpu.sync_copy(tgt_hbm.at[0, b, pl.ds(k0, KH)], tgt0_v)
    pltpu.sync_copy(tgt_hbm.at[1, b, pl.ds(k0, KH)], tgt1_v)

    base0 = b * (C * HW)
    for i in range(NCHUNK):
        ind_c = ind_v[pl.ds(i * LANES, LANES)]
        flat0 = (base0 + (ind_c & -4096)
                 + ((ind_c & (3 << 7)) << 3)
                 + ((ind_c & (7 << 9)) >> 2)
                 + (ind_c & 127))
        g = i // 8
        o = (i % 8) * LANES
        idx_v[g, pl.ds(o, LANES)] = flat0
        idx_v[2 + g, pl.ds(o, LANES)] = flat0 + HW

    copies = [
        pltpu.async_copy(
            table_hbm.at[idx_v.at[j]],
            vals_v.at[pl.ds(j * GATHER_GROUP, GATHER_GROUP)],
            sem,
        )
        for j in range(NGROUP)
    ]
    for cp in copies:
        cp.wait()

    loss_acc = jnp.zeros((LANES,), jnp.float32)
    num_acc = jnp.zeros((LANES,), jnp.float32)
    for i in range(NCHUNK):
        sl = pl.ds(i * LANES, LANES)
        v0 = vals_v[sl]
        v1 = vals_v[pl.ds(KH + i * LANES, LANES)]
        m = mask_v[sl]
        d0 = (v0 - tgt0_v[sl]) * m
        d1 = (v1 - tgt1_v[sl]) * m
        a0 = jnp.abs(d0)
        a1 = jnp.abs(d1)
        e0 = jnp.where(a0 < 1.0, 0.5 * d0 * d0, a0 - 0.5)
        e1 = jnp.where(a1 < 1.0, 0.5 * d1 * d1, a1 - 0.5)
        loss_acc = loss_acc + e0 + e1
        num_acc = num_acc + m

    zero = jnp.zeros((LANES,), jnp.float32)
    acc_v[pl.ds(0, LANES)] = loss_acc
    acc_v[pl.ds(LANES, LANES)] = num_acc
    for z in range(2, 8):
        acc_v[pl.ds(z * LANES, LANES)] = zero
    pltpu.sync_copy(acc_v, part_out.at[wid])


@jax.jit
def _reg_loss_sc(table, ind_pad, maskf_pad, tgt_pad):
    mesh = plsc.VectorSubcoreMesh(core_axis_name="c", subcore_axis_name="s")
    k = functools.partial(
        pl.kernel,
        mesh=mesh,
        out_type=jax.ShapeDtypeStruct((32, 128), jnp.float32),
        scratch_types=[
            pltpu.VMEM((KH,), jnp.int32),
            pltpu.VMEM((KH,), jnp.float32),
            pltpu.VMEM((KH,), jnp.float32),
            pltpu.VMEM((KH,), jnp.float32),
            pltpu.VMEM((NGROUP, GATHER_GROUP), jnp.int32),
            pltpu.VMEM((ELEMS_PER_TILE,), jnp.float32),
            pltpu.VMEM((128,), jnp.float32),
            pltpu.SemaphoreType.DMA,
        ],
    )(_sc_body)
    part = k(table, ind_pad, maskf_pad, tgt_pad)
    num = part[:, LANES:2 * LANES].sum()
    return part[:, :LANES].sum() / (num + 0.0001)


def kernel(output, mask, ind, target):
    table = jnp.transpose(
        output.reshape(B, C, 512 // 8, 8, 512 // 128, 128),
        (0, 1, 2, 4, 3, 5)).reshape(B * C * HW)
    ind_pad = jnp.pad(ind.astype(jnp.int32), ((0, 0), (0, KPAD - K)))
    maskf_pad = jnp.pad(mask.astype(jnp.float32), ((0, 0), (0, KPAD - K)))
    tgt_pad = jnp.pad(jnp.transpose(target, (2, 0, 1)),
                      ((0, 0), (0, 0), (0, KPAD - K)))
    return _reg_loss_sc(table, ind_pad, maskf_pad, tgt_pad)


# speedup vs baseline: 3.9017x; 1.0037x over previous
import functools

import jax
import jax.numpy as jnp
from jax import lax
from jax.experimental import pallas as pl
from jax.experimental.pallas import tpu as pltpu
from jax.experimental.pallas import tpu_sc as plsc

B = 16
C = 2
HW = 512 * 512
K = 500
KPAD = 512
KH = KPAD // 2
LANES = 16
NCHUNK = KH // LANES
ELEMS_PER_TILE = 2 * KH
GATHER_GROUP = 128
NGROUP = ELEMS_PER_TILE // GATHER_GROUP


def _sc_body(table_hbm, ind_hbm, maskf_hbm, tgt_hbm,
             part_out,
             ind_v, mask_v, tgt0_v, tgt1_v, idx_v, vals_v, acc_v, sem):
    nc = 2
    wid = lax.axis_index("s") * nc + lax.axis_index("c")
    b = wid // 2
    k0 = (wid % 2) * KH

    pltpu.sync_copy(ind_hbm.at[b, pl.ds(k0, KH)], ind_v)
    pltpu.sync_copy(maskf_hbm.at[b, pl.ds(k0, KH)], mask_v)
    pltpu.sync_copy(tgt_hbm.at[0, b, pl.ds(k0, KH)], tgt0_v)
    pltpu.sync_copy(tgt_hbm.at[1, b, pl.ds(k0, KH)], tgt1_v)

    base0 = b * (C * HW)

    def _idx_body(i, carry):
        off = pl.multiple_of(i * LANES, LANES)
        ind_c = ind_v[pl.ds(off, LANES)]
        flat0 = (base0 + (ind_c & -4096)
                 + ((ind_c & (3 << 7)) << 3)
                 + ((ind_c & (7 << 9)) >> 2)
                 + (ind_c & 127))
        g = i // 8
        o = pl.multiple_of((i % 8) * LANES, LANES)
        idx_v[g, pl.ds(o, LANES)] = flat0
        idx_v[2 + g, pl.ds(o, LANES)] = flat0 + HW
        return carry

    lax.fori_loop(0, NCHUNK, _idx_body, 0)

    copies = [
        pltpu.async_copy(
            table_hbm.at[idx_v.at[j]],
            vals_v.at[pl.ds(j * GATHER_GROUP, GATHER_GROUP)],
            sem,
        )
        for j in range(NGROUP)
    ]
    for cp in copies:
        cp.wait()

    def _loss_body(i, carry):
        l_acc, n_acc = carry
        off = pl.multiple_of(i * LANES, LANES)
        sl = pl.ds(off, LANES)
        v0 = vals_v[sl]
        v1 = vals_v[pl.ds(off + KH, LANES)]
        m = mask_v[sl]
        d0 = (v0 - tgt0_v[sl]) * m
        d1 = (v1 - tgt1_v[sl]) * m
        a0 = jnp.abs(d0)
        a1 = jnp.abs(d1)
        e0 = jnp.where(a0 < 1.0, 0.5 * d0 * d0, a0 - 0.5)
        e1 = jnp.where(a1 < 1.0, 0.5 * d1 * d1, a1 - 0.5)
        return l_acc + e0 + e1, n_acc + m

    loss_acc, num_acc = lax.fori_loop(
        0, NCHUNK, _loss_body,
        (jnp.zeros((LANES,), jnp.float32), jnp.zeros((LANES,), jnp.float32)))

    zero = jnp.zeros((LANES,), jnp.float32)
    acc_v[pl.ds(0, LANES)] = loss_acc
    acc_v[pl.ds(LANES, LANES)] = num_acc
    for z in range(2, 8):
        acc_v[pl.ds(z * LANES, LANES)] = zero
    pltpu.sync_copy(acc_v, part_out.at[wid])


@jax.jit
def _reg_loss_sc(table, ind_pad, maskf_pad, tgt_pad):
    mesh = plsc.VectorSubcoreMesh(core_axis_name="c", subcore_axis_name="s")
    k = functools.partial(
        pl.kernel,
        mesh=mesh,
        out_type=jax.ShapeDtypeStruct((32, 128), jnp.float32),
        scratch_types=[
            pltpu.VMEM((KH,), jnp.int32),
            pltpu.VMEM((KH,), jnp.float32),
            pltpu.VMEM((KH,), jnp.float32),
            pltpu.VMEM((KH,), jnp.float32),
            pltpu.VMEM((NGROUP, GATHER_GROUP), jnp.int32),
            pltpu.VMEM((ELEMS_PER_TILE,), jnp.float32),
            pltpu.VMEM((128,), jnp.float32),
            pltpu.SemaphoreType.DMA,
        ],
    )(_sc_body)
    part = k(table, ind_pad, maskf_pad, tgt_pad)
    num = part[:, LANES:2 * LANES].sum()
    return part[:, :LANES].sum() / (num + 0.0001)


def kernel(output, mask, ind, target):
    table = jnp.transpose(
        output.reshape(B, C, 512 // 8, 8, 512 // 128, 128),
        (0, 1, 2, 4, 3, 5)).reshape(B * C * HW)
    ind_pad = jnp.pad(ind.astype(jnp.int32), ((0, 0), (0, KPAD - K)))
    maskf_pad = jnp.pad(mask.astype(jnp.float32), ((0, 0), (0, KPAD - K)))
    tgt_pad = jnp.pad(jnp.transpose(target, (2, 0, 1)),
                      ((0, 0), (0, 0), (0, KPAD - K)))
    return _reg_loss_sc(table, ind_pad, maskf_pad, tgt_pad)


# speedup vs baseline: 4.1420x; 1.0616x over previous
import functools

import jax
import jax.numpy as jnp
from jax import lax
from jax.experimental import pallas as pl
from jax.experimental.pallas import tpu as pltpu
from jax.experimental.pallas import tpu_sc as plsc

B = 16
C = 2
HW = 512 * 512
K = 500
KPAD = 512
LANES = 16
KH = KPAD // 2
NCHUNK = KH // LANES
ELEMS_PER_TILE = 2 * KH
GATHER_GROUP = 128
NGROUP = ELEMS_PER_TILE // GATHER_GROUP


def _sc_body(table_hbm, ind_hbm, buf_hbm,
             part_out,
             ind_v, mask_v, tgt_v, idx_v, vals_v, acc_v, sem, sem2):
    nc = 2
    wid = lax.axis_index("s") * nc + lax.axis_index("c")
    b = wid // 2
    half = wid % 2
    k0 = half * KH

    pltpu.sync_copy(ind_hbm.at[b, pl.ds(k0, KH)], ind_v)
    mcp = pltpu.async_copy(buf_hbm.at[0, b, pl.ds(k0, KH)], mask_v, sem2)
    tcp = pltpu.async_copy(buf_hbm.at[1 + half, b], tgt_v, sem2)

    base0 = b * (C * HW)

    def _idx_body(li, carry):
        ind_c = ind_v[pl.ds(pl.multiple_of(li * LANES, LANES), LANES)]
        flat0 = (base0 + (ind_c & -4096)
                 + ((ind_c & (3 << 7)) << 3)
                 + ((ind_c & (7 << 9)) >> 2)
                 + (ind_c & 127))
        g = li // 8
        o = pl.multiple_of((li % 8) * LANES, LANES)
        idx_v[g, pl.ds(o, LANES)] = flat0
        idx_v[2 + g, pl.ds(o, LANES)] = flat0 + HW
        return carry

    lax.fori_loop(0, NCHUNK, _idx_body, 0)

    copies = [
        pltpu.async_copy(
            table_hbm.at[idx_v.at[j]],
            vals_v.at[pl.ds(j * GATHER_GROUP, GATHER_GROUP)],
            sem,
        )
        for j in range(NGROUP)
    ]
    mcp.wait()
    tcp.wait()
    for cp in copies:
        cp.wait()

    iota = lax.iota(jnp.int32, LANES)
    ev = (iota << 1) & 15
    od = ev | 1
    lo8 = iota < 8
    zero = jnp.zeros((LANES,), jnp.float32)

    def _vgather(vec, idx):
        return lax.gather(
            vec, idx[:, None],
            dimension_numbers=lax.GatherDimensionNumbers(
                offset_dims=(), collapsed_slice_dims=(0,),
                start_index_map=(0,)),
            slice_sizes=(1,),
            mode=lax.GatherScatterMode.PROMISE_IN_BOUNDS)

    def _loss_body(li, carry):
        l_acc, n_acc = carry
        lsl = pl.ds(pl.multiple_of(li * LANES, LANES), LANES)
        v0 = vals_v[lsl]
        v1 = vals_v[pl.ds(pl.multiple_of(li * LANES + KH, LANES), LANES)]
        m = mask_v[lsl]
        ta = tgt_v[pl.ds(pl.multiple_of(li * 2 * LANES, LANES), LANES)]
        tb = tgt_v[pl.ds(pl.multiple_of(li * 2 * LANES + LANES, LANES), LANES)]
        t0 = jnp.where(lo8, _vgather(ta, ev), _vgather(tb, ev))
        t1 = jnp.where(lo8, _vgather(ta, od), _vgather(tb, od))
        d0 = (v0 - t0) * m
        d1 = (v1 - t1) * m
        a0 = jnp.abs(d0)
        a1 = jnp.abs(d1)
        e0 = jnp.where(a0 < 1.0, 0.5 * d0 * d0, a0 - 0.5)
        e1 = jnp.where(a1 < 1.0, 0.5 * d1 * d1, a1 - 0.5)
        return l_acc + e0 + e1, n_acc + m

    loss_acc, num_acc = lax.fori_loop(
        0, NCHUNK, _loss_body, (zero, zero))

    acc_v[pl.ds(0, LANES)] = loss_acc
    acc_v[pl.ds(LANES, LANES)] = num_acc
    for z in range(2, 8):
        acc_v[pl.ds(z * LANES, LANES)] = zero
    pltpu.sync_copy(acc_v, part_out.at[wid])


@jax.jit
def _reg_loss_sc(table, ind_pad, buf):
    mesh = plsc.VectorSubcoreMesh(core_axis_name="c", subcore_axis_name="s")
    k = functools.partial(
        pl.kernel,
        mesh=mesh,
        out_type=jax.ShapeDtypeStruct((32, 128), jnp.float32),
        scratch_types=[
            pltpu.VMEM((KH,), jnp.int32),
            pltpu.VMEM((KH,), jnp.float32),
            pltpu.VMEM((2 * KH,), jnp.float32),
            pltpu.VMEM((NGROUP, GATHER_GROUP), jnp.int32),
            pltpu.VMEM((ELEMS_PER_TILE,), jnp.float32),
            pltpu.VMEM((128,), jnp.float32),
            pltpu.SemaphoreType.DMA,
            pltpu.SemaphoreType.DMA,
        ],
    )(_sc_body)
    part = k(table, ind_pad, buf)
    num = part[:, LANES:2 * LANES].sum()
    return part[:, :LANES].sum() / (num + 0.0001)


def kernel(output, mask, ind, target):
    table = jnp.transpose(
        output.reshape(B, C, 512 // 8, 8, 512 // 128, 128),
        (0, 1, 2, 4, 3, 5)).reshape(B * C * HW)
    ind_pad = jnp.pad(ind.astype(jnp.int32), ((0, 0), (0, KPAD - K)))
    maskf = jnp.pad(mask.astype(jnp.float32)[None],
                    ((0, 0), (0, 0), (0, KPAD - K)))
    tgt = jnp.pad(target.reshape(B, 2 * K), ((0, 0), (0, 2 * (KPAD - K))))
    buf = jnp.concatenate(
        [maskf, jnp.transpose(tgt.reshape(B, 2, KPAD), (1, 0, 2))], axis=0)
    return _reg_loss_sc(table, ind_pad, buf)


# speedup vs baseline: 4.1696x; 1.0067x over previous
import functools

import jax
import jax.numpy as jnp
from jax import lax
from jax.experimental import pallas as pl
from jax.experimental.pallas import tpu as pltpu
from jax.experimental.pallas import tpu_sc as plsc

B = 16
C = 2
HW = 512 * 512
K = 500
KPAD = 512
LANES = 16
KH = KPAD // 2
NCHUNK = KH // LANES
ELEMS_PER_TILE = 2 * KH
GATHER_GROUP = 128
NGROUP = ELEMS_PER_TILE // GATHER_GROUP


def _sc_body(table_hbm, ind_hbm, buf_hbm,
             part_out,
             ind_v, mask_v, tgt_v, idx_v, vals_v, acc_v, sem, sem2, sem3):
    nc = 2
    wid = lax.axis_index("s") * nc + lax.axis_index("c")
    b = wid // 2
    half = wid % 2
    k0 = half * KH

    pltpu.sync_copy(ind_hbm.at[b, pl.ds(k0, KH)], ind_v)
    mcp = pltpu.async_copy(buf_hbm.at[b, pl.ds(k0, KH)], mask_v, sem2)
    tcp = pltpu.async_copy(
        buf_hbm.at[b, pl.ds(KPAD + half * 2 * KH, 2 * KH)], tgt_v, sem2)

    base0 = b * (C * HW)

    def _idx_body(li, carry):
        ind_c = ind_v[pl.ds(pl.multiple_of(li * LANES, LANES), LANES)]
        flat0 = (base0 + (ind_c & -4096)
                 + ((ind_c & (3 << 7)) << 3)
                 + ((ind_c & (7 << 9)) >> 2)
                 + (ind_c & 127))
        g = li // 8
        o = pl.multiple_of((li % 8) * LANES, LANES)
        idx_v[g, pl.ds(o, LANES)] = flat0
        idx_v[2 + g, pl.ds(o, LANES)] = flat0 + HW
        return carry

    def _fire(j, s):
        return pltpu.async_copy(
            table_hbm.at[idx_v.at[j]],
            vals_v.at[pl.ds(j * GATHER_GROUP, GATHER_GROUP)],
            s,
        )

    lax.fori_loop(0, NCHUNK // 2, _idx_body, 0)
    ga = [_fire(0, sem), _fire(2, sem)]
    lax.fori_loop(NCHUNK // 2, NCHUNK, _idx_body, 0)
    gb = [_fire(1, sem3), _fire(3, sem3)]
    mcp.wait()
    tcp.wait()

    iota = lax.iota(jnp.int32, LANES)
    ev = (iota << 1) & 15
    od = ev | 1
    lo8 = iota < 8
    zero = jnp.zeros((LANES,), jnp.float32)

    def _vgather(vec, idx):
        return lax.gather(
            vec, idx[:, None],
            dimension_numbers=lax.GatherDimensionNumbers(
                offset_dims=(), collapsed_slice_dims=(0,),
                start_index_map=(0,)),
            slice_sizes=(1,),
            mode=lax.GatherScatterMode.PROMISE_IN_BOUNDS)

    def _loss_body(li, carry):
        l_acc, n_acc = carry
        lsl = pl.ds(pl.multiple_of(li * LANES, LANES), LANES)
        v0 = vals_v[lsl]
        v1 = vals_v[pl.ds(pl.multiple_of(li * LANES + KH, LANES), LANES)]
        m = mask_v[lsl]
        ta = tgt_v[pl.ds(pl.multiple_of(li * 2 * LANES, LANES), LANES)]
        tb = tgt_v[pl.ds(pl.multiple_of(li * 2 * LANES + LANES, LANES), LANES)]
        t0 = jnp.where(lo8, _vgather(ta, ev), _vgather(tb, ev))
        t1 = jnp.where(lo8, _vgather(ta, od), _vgather(tb, od))
        d0 = (v0 - t0) * m
        d1 = (v1 - t1) * m
        a0 = jnp.abs(d0)
        a1 = jnp.abs(d1)
        e0 = jnp.where(a0 < 1.0, 0.5 * d0 * d0, a0 - 0.5)
        e1 = jnp.where(a1 < 1.0, 0.5 * d1 * d1, a1 - 0.5)
        return l_acc + e0 + e1, n_acc + m

    for cp in ga:
        cp.wait()
    acc_half = lax.fori_loop(0, NCHUNK // 2, _loss_body, (zero, zero))
    for cp in gb:
        cp.wait()
    loss_acc, num_acc = lax.fori_loop(
        NCHUNK // 2, NCHUNK, _loss_body, acc_half)

    acc_v[pl.ds(0, LANES)] = loss_acc
    acc_v[pl.ds(LANES, LANES)] = num_acc
    for z in range(2, 8):
        acc_v[pl.ds(z * LANES, LANES)] = zero
    pltpu.sync_copy(acc_v, part_out.at[wid])


@jax.jit
def _reg_loss_sc(table, ind_pad, buf):
    mesh = plsc.VectorSubcoreMesh(core_axis_name="c", subcore_axis_name="s")
    k = functools.partial(
        pl.kernel,
        mesh=mesh,
        out_type=jax.ShapeDtypeStruct((32, 128), jnp.float32),
        scratch_types=[
            pltpu.VMEM((KH,), jnp.int32),
            pltpu.VMEM((KH,), jnp.float32),
            pltpu.VMEM((2 * KH,), jnp.float32),
            pltpu.VMEM((NGROUP, GATHER_GROUP), jnp.int32),
            pltpu.VMEM((ELEMS_PER_TILE,), jnp.float32),
            pltpu.VMEM((128,), jnp.float32),
            pltpu.SemaphoreType.DMA,
            pltpu.SemaphoreType.DMA,
            pltpu.SemaphoreType.DMA,
        ],
    )(_sc_body)
    part = k(table, ind_pad, buf)
    num = part[:, LANES:2 * LANES].sum()
    return part[:, :LANES].sum() / (num + 0.0001)


def kernel(output, mask, ind, target):
    table = jnp.transpose(
        output.reshape(B, C, 512 // 8, 8, 512 // 128, 128),
        (0, 1, 2, 4, 3, 5)).reshape(B * C * HW)
    ind_pad = jnp.pad(ind.astype(jnp.int32), ((0, 0), (0, KPAD - K)))
    maskf = jnp.pad(mask.astype(jnp.float32), ((0, 0), (0, KPAD - K)))
    tgt = jnp.pad(target.reshape(B, 2 * K), ((0, 0), (0, 2 * (KPAD - K))))
    buf = jnp.concatenate([maskf, tgt], axis=1)
    return _reg_loss_sc(table, ind_pad, buf)


# speedup vs baseline: 4.1782x; 1.0021x over previous
import functools

import jax
import jax.numpy as jnp
from jax import lax
from jax.experimental import pallas as pl
from jax.experimental.pallas import tpu as pltpu
from jax.experimental.pallas import tpu_sc as plsc

B = 16
C = 2
HW = 512 * 512
K = 500
KPAD = 512
LANES = 16
KH = KPAD // 2
NCHUNK = KH // LANES
ELEMS_PER_TILE = 2 * KH
GATHER_GROUP = 128
NGROUP = ELEMS_PER_TILE // GATHER_GROUP


def _sc_body(table_hbm, ind_hbm, buf_hbm,
             part_out,
             ind_v, mask_v, tgt_v, idx_v, vals_v, acc_v, sem, sem2, sem3):
    nc = 2
    wid = lax.axis_index("s") * nc + lax.axis_index("c")
    b = wid // 2
    half = wid % 2
    k0 = half * KH

    mcp = pltpu.async_copy(buf_hbm.at[b, pl.ds(k0, KH)], mask_v, sem2)
    tcp = pltpu.async_copy(
        buf_hbm.at[b, pl.ds(KPAD + half * 2 * KH, 2 * KH)], tgt_v, sem2)
    pltpu.sync_copy(ind_hbm.at[b, pl.ds(k0, KH)], ind_v)

    base0 = b * (C * HW)

    def _idx_body(li, carry):
        ind_c = ind_v[pl.ds(pl.multiple_of(li * LANES, LANES), LANES)]
        flat0 = (base0 + (ind_c & -4096)
                 + ((ind_c & (3 << 7)) << 3)
                 + ((ind_c & (7 << 9)) >> 2)
                 + (ind_c & 127))
        g = li // 8
        o = pl.multiple_of((li % 8) * LANES, LANES)
        idx_v[g, pl.ds(o, LANES)] = flat0
        idx_v[2 + g, pl.ds(o, LANES)] = flat0 + HW
        return carry

    def _fire(j, s):
        return pltpu.async_copy(
            table_hbm.at[idx_v.at[j]],
            vals_v.at[pl.ds(j * GATHER_GROUP, GATHER_GROUP)],
            s,
        )

    lax.fori_loop(0, NCHUNK // 2, _idx_body, 0)
    ga = [_fire(0, sem), _fire(2, sem)]
    lax.fori_loop(NCHUNK // 2, NCHUNK, _idx_body, 0)
    gb = [_fire(1, sem3), _fire(3, sem3)]
    mcp.wait()
    tcp.wait()

    iota = lax.iota(jnp.int32, LANES)
    ev = (iota << 1) & 15
    od = ev | 1
    lo8 = iota < 8
    zero = jnp.zeros((LANES,), jnp.float32)

    def _vgather(vec, idx):
        return lax.gather(
            vec, idx[:, None],
            dimension_numbers=lax.GatherDimensionNumbers(
                offset_dims=(), collapsed_slice_dims=(0,),
                start_index_map=(0,)),
            slice_sizes=(1,),
            mode=lax.GatherScatterMode.PROMISE_IN_BOUNDS)

    def _loss_body(li, carry):
        l_acc, n_acc = carry
        lsl = pl.ds(pl.multiple_of(li * LANES, LANES), LANES)
        v0 = vals_v[lsl]
        v1 = vals_v[pl.ds(pl.multiple_of(li * LANES + KH, LANES), LANES)]
        m = mask_v[lsl]
        ta = tgt_v[pl.ds(pl.multiple_of(li * 2 * LANES, LANES), LANES)]
        tb = tgt_v[pl.ds(pl.multiple_of(li * 2 * LANES + LANES, LANES), LANES)]
        t0 = jnp.where(lo8, _vgather(ta, ev), _vgather(tb, ev))
        t1 = jnp.where(lo8, _vgather(ta, od), _vgather(tb, od))
        d0 = (v0 - t0) * m
        d1 = (v1 - t1) * m
        a0 = jnp.abs(d0)
        a1 = jnp.abs(d1)
        e0 = jnp.where(a0 < 1.0, 0.5 * d0 * d0, a0 - 0.5)
        e1 = jnp.where(a1 < 1.0, 0.5 * d1 * d1, a1 - 0.5)
        return l_acc + e0 + e1, n_acc + m

    for cp in ga:
        cp.wait()
    acc_half = lax.fori_loop(0, NCHUNK // 2, _loss_body, (zero, zero))
    for cp in gb:
        cp.wait()
    loss_acc, num_acc = lax.fori_loop(
        NCHUNK // 2, NCHUNK, _loss_body, acc_half)

    acc_v[pl.ds(0, LANES)] = loss_acc
    acc_v[pl.ds(LANES, LANES)] = num_acc
    for z in range(2, 8):
        acc_v[pl.ds(z * LANES, LANES)] = zero
    pltpu.sync_copy(acc_v, part_out.at[wid])


@jax.jit
def _reg_loss_sc(table, ind_pad, buf):
    mesh = plsc.VectorSubcoreMesh(core_axis_name="c", subcore_axis_name="s")
    k = functools.partial(
        pl.kernel,
        mesh=mesh,
        out_type=jax.ShapeDtypeStruct((32, 128), jnp.float32),
        scratch_types=[
            pltpu.VMEM((KH,), jnp.int32),
            pltpu.VMEM((KH,), jnp.float32),
            pltpu.VMEM((2 * KH,), jnp.float32),
            pltpu.VMEM((NGROUP, GATHER_GROUP), jnp.int32),
            pltpu.VMEM((ELEMS_PER_TILE,), jnp.float32),
            pltpu.VMEM((128,), jnp.float32),
            pltpu.SemaphoreType.DMA,
            pltpu.SemaphoreType.DMA,
            pltpu.SemaphoreType.DMA,
        ],
    )(_sc_body)
    part = k(table, ind_pad, buf)
    num = part[:, LANES:2 * LANES].sum()
    return part[:, :LANES].sum() / (num + 0.0001)


def kernel(output, mask, ind, target):
    table = jnp.transpose(
        output.reshape(B, C, 512 // 8, 8, 512 // 128, 128),
        (0, 1, 2, 4, 3, 5)).reshape(B * C * HW)
    ind_pad = jnp.pad(ind.astype(jnp.int32), ((0, 0), (0, KPAD - K)))
    maskf = jnp.pad(mask.astype(jnp.float32), ((0, 0), (0, KPAD - K)))
    tgt = jnp.pad(target.reshape(B, 2 * K), ((0, 0), (0, 2 * (KPAD - K))))
    buf = jnp.concatenate([maskf, tgt], axis=1)
    return _reg_loss_sc(table, ind_pad, buf)


# speedup vs baseline: 4.3233x; 1.0347x over previous
import functools

import jax
import jax.numpy as jnp
from jax import lax
from jax.experimental import pallas as pl
from jax.experimental.pallas import tpu as pltpu
from jax.experimental.pallas import tpu_sc as plsc

B = 16
C = 2
HW = 512 * 512
K = 500
KPAD = 512
LANES = 16
KH = KPAD
NCHUNK = KH // LANES
ELEMS_PER_TILE = 2 * KH
GATHER_GROUP = 128
NGROUP = ELEMS_PER_TILE // GATHER_GROUP


def _sc_body(table_hbm, ind_hbm, buf_hbm,
             part_out,
             ind_v, mask_v, tgt_v, idx_v, vals_v, acc_v, sem, sem2, sem3):
    wid = lax.axis_index("s")
    b = wid
    k0 = 0

    mcp = pltpu.async_copy(buf_hbm.at[b, pl.ds(k0, KH)], mask_v, sem2)
    tcp = pltpu.async_copy(
        buf_hbm.at[b, pl.ds(KPAD, 2 * KH)], tgt_v, sem2)
    pltpu.sync_copy(ind_hbm.at[b, pl.ds(k0, KH)], ind_v)

    base0 = b * (C * HW)

    def _idx_body(li, carry):
        ind_c = ind_v[pl.ds(pl.multiple_of(li * LANES, LANES), LANES)]
        flat0 = (base0 + (ind_c & -4096)
                 + ((ind_c & (3 << 7)) << 3)
                 + ((ind_c & (7 << 9)) >> 2)
                 + (ind_c & 127))
        g = li // 8
        o = pl.multiple_of((li % 8) * LANES, LANES)
        idx_v[g, pl.ds(o, LANES)] = flat0
        idx_v[NGROUP // 2 + g, pl.ds(o, LANES)] = flat0 + HW
        return carry

    def _fire(j, s):
        return pltpu.async_copy(
            table_hbm.at[idx_v.at[j]],
            vals_v.at[pl.ds(j * GATHER_GROUP, GATHER_GROUP)],
            s,
        )

    lax.fori_loop(0, NCHUNK // 2, _idx_body, 0)
    ga = [_fire(0, sem), _fire(1, sem), _fire(4, sem), _fire(5, sem)]
    lax.fori_loop(NCHUNK // 2, NCHUNK, _idx_body, 0)
    gb = [_fire(2, sem3), _fire(3, sem3), _fire(6, sem3), _fire(7, sem3)]
    mcp.wait()
    tcp.wait()

    iota = lax.iota(jnp.int32, LANES)
    ev = (iota << 1) & 15
    od = ev | 1
    lo8 = iota < 8
    zero = jnp.zeros((LANES,), jnp.float32)

    def _vgather(vec, idx):
        return lax.gather(
            vec, idx[:, None],
            dimension_numbers=lax.GatherDimensionNumbers(
                offset_dims=(), collapsed_slice_dims=(0,),
                start_index_map=(0,)),
            slice_sizes=(1,),
            mode=lax.GatherScatterMode.PROMISE_IN_BOUNDS)

    def _loss_body(li, carry):
        l_acc, n_acc = carry
        lsl = pl.ds(pl.multiple_of(li * LANES, LANES), LANES)
        v0 = vals_v[lsl]
        v1 = vals_v[pl.ds(pl.multiple_of(li * LANES + KH, LANES), LANES)]
        m = mask_v[lsl]
        ta = tgt_v[pl.ds(pl.multiple_of(li * 2 * LANES, LANES), LANES)]
        tb = tgt_v[pl.ds(pl.multiple_of(li * 2 * LANES + LANES, LANES), LANES)]
        t0 = jnp.where(lo8, _vgather(ta, ev), _vgather(tb, ev))
        t1 = jnp.where(lo8, _vgather(ta, od), _vgather(tb, od))
        d0 = (v0 - t0) * m
        d1 = (v1 - t1) * m
        a0 = jnp.abs(d0)
        a1 = jnp.abs(d1)
        e0 = jnp.where(a0 < 1.0, 0.5 * d0 * d0, a0 - 0.5)
        e1 = jnp.where(a1 < 1.0, 0.5 * d1 * d1, a1 - 0.5)
        return l_acc + e0 + e1, n_acc + m

    for cp in ga:
        cp.wait()
    acc_half = lax.fori_loop(0, NCHUNK // 2, _loss_body, (zero, zero))
    for cp in gb:
        cp.wait()
    loss_acc, num_acc = lax.fori_loop(
        NCHUNK // 2, NCHUNK, _loss_body, acc_half)

    acc_v[pl.ds(0, LANES)] = loss_acc
    acc_v[pl.ds(LANES, LANES)] = num_acc
    for z in range(2, 8):
        acc_v[pl.ds(z * LANES, LANES)] = zero
    pltpu.sync_copy(acc_v, part_out.at[wid])


@jax.jit
def _reg_loss_sc(table, ind_pad, buf):
    mesh = plsc.VectorSubcoreMesh(core_axis_name="c", subcore_axis_name="s", num_cores=1)
    k = functools.partial(
        pl.kernel,
        mesh=mesh,
        out_type=jax.ShapeDtypeStruct((16, 128), jnp.float32),
        scratch_types=[
            pltpu.VMEM((KH,), jnp.int32),
            pltpu.VMEM((KH,), jnp.float32),
            pltpu.VMEM((2 * KH,), jnp.float32),
            pltpu.VMEM((NGROUP, GATHER_GROUP), jnp.int32),
            pltpu.VMEM((ELEMS_PER_TILE,), jnp.float32),
            pltpu.VMEM((128,), jnp.float32),
            pltpu.SemaphoreType.DMA,
            pltpu.SemaphoreType.DMA,
            pltpu.SemaphoreType.DMA,
        ],
    )(_sc_body)
    part = k(table, ind_pad, buf)
    num = part[:, LANES:2 * LANES].sum()
    return part[:, :LANES].sum() / (num + 0.0001)


def kernel(output, mask, ind, target):
    table = jnp.transpose(
        output.reshape(B, C, 512 // 8, 8, 512 // 128, 128),
        (0, 1, 2, 4, 3, 5)).reshape(B * C * HW)
    ind_pad = jnp.pad(ind.astype(jnp.int32), ((0, 0), (0, KPAD - K)))
    maskf = jnp.pad(mask.astype(jnp.float32), ((0, 0), (0, KPAD - K)))
    tgt = jnp.pad(target.reshape(B, 2 * K), ((0, 0), (0, 2 * (KPAD - K))))
    buf = jnp.concatenate([maskf, tgt], axis=1)
    return _reg_loss_sc(table, ind_pad, buf)


# speedup vs baseline: 4.3923x; 1.0160x over previous
import functools

import jax
import jax.numpy as jnp
from jax import lax
from jax.experimental import pallas as pl
from jax.experimental.pallas import tpu as pltpu
from jax.experimental.pallas import tpu_sc as plsc

B = 16
C = 2
HW = 512 * 512
K = 500
KPAD = 512
LANES = 16
KH = KPAD
NCHUNK = KH // LANES
ELEMS_PER_TILE = 2 * KH
GATHER_GROUP = 128
NGROUP = ELEMS_PER_TILE // GATHER_GROUP


def _sc_body(table_hbm, ind_hbm, buf_hbm,
             part_out,
             ind_v, mask_v, tgt_v, idx_v, vals_v, acc_v, sem, sem2, sem3):
    wid = lax.axis_index("s")
    b = wid
    k0 = 0

    mcp = pltpu.async_copy(buf_hbm.at[b, pl.ds(k0, KH)], mask_v, sem2)
    tcp = pltpu.async_copy(
        buf_hbm.at[b, pl.ds(KPAD, 2 * KH)], tgt_v, sem2)
    pltpu.sync_copy(ind_hbm.at[b, pl.ds(k0, KH)], ind_v)

    base0 = b * (C * HW)

    def _idx_body(li, carry):
        ind_c = ind_v[pl.ds(pl.multiple_of(li * LANES, LANES), LANES)]
        flat0 = (base0 + (ind_c & -4096)
                 + ((ind_c & (3 << 7)) << 3)
                 + ((ind_c & (7 << 9)) >> 2)
                 + (ind_c & 127))
        g = li // 8
        o = pl.multiple_of((li % 8) * LANES, LANES)
        idx_v[g, pl.ds(o, LANES)] = flat0
        idx_v[NGROUP // 2 + g, pl.ds(o, LANES)] = flat0 + HW
        return carry

    def _fire(j, s):
        return pltpu.async_copy(
            table_hbm.at[idx_v.at[j]],
            vals_v.at[pl.ds(j * GATHER_GROUP, GATHER_GROUP)],
            s,
        )

    lax.fori_loop(0, NCHUNK // 2, _idx_body, 0)
    ga = [_fire(0, sem), _fire(1, sem), _fire(4, sem), _fire(5, sem)]
    lax.fori_loop(NCHUNK // 2, NCHUNK, _idx_body, 0)
    gb = [_fire(2, sem3), _fire(3, sem3), _fire(6, sem3), _fire(7, sem3)]
    mcp.wait()
    tcp.wait()

    zero = jnp.zeros((LANES,), jnp.float32)

    def _loss_body(li, carry):
        l_acc, n_acc = carry
        lsl = pl.ds(pl.multiple_of(li * LANES, LANES), LANES)
        v0 = vals_v[lsl]
        v1 = vals_v[pl.ds(pl.multiple_of(li * LANES + KH, LANES), LANES)]
        m = mask_v[lsl]
        tbase = (li >> 3) * 256 + (li & 7) * LANES
        t0 = tgt_v[pl.ds(pl.multiple_of(tbase, LANES), LANES)]
        t1 = tgt_v[pl.ds(pl.multiple_of(tbase + 128, LANES), LANES)]
        d0 = (v0 - t0) * m
        d1 = (v1 - t1) * m
        a0 = jnp.abs(d0)
        a1 = jnp.abs(d1)
        e0 = jnp.where(a0 < 1.0, 0.5 * d0 * d0, a0 - 0.5)
        e1 = jnp.where(a1 < 1.0, 0.5 * d1 * d1, a1 - 0.5)
        return l_acc + e0 + e1, n_acc + m

    for cp in ga:
        cp.wait()
    acc_half = lax.fori_loop(0, NCHUNK // 2, _loss_body, (zero, zero))
    for cp in gb:
        cp.wait()
    loss_acc, num_acc = lax.fori_loop(
        NCHUNK // 2, NCHUNK, _loss_body, acc_half)

    acc_v[pl.ds(0, LANES)] = loss_acc
    acc_v[pl.ds(LANES, LANES)] = num_acc
    for z in range(2, 8):
        acc_v[pl.ds(z * LANES, LANES)] = zero
    pltpu.sync_copy(acc_v, part_out.at[wid])


@jax.jit
def _reg_loss_sc(table, ind_pad, buf):
    mesh = plsc.VectorSubcoreMesh(core_axis_name="c", subcore_axis_name="s", num_cores=1)
    k = functools.partial(
        pl.kernel,
        mesh=mesh,
        out_type=jax.ShapeDtypeStruct((16, 128), jnp.float32),
        scratch_types=[
            pltpu.VMEM((KH,), jnp.int32),
            pltpu.VMEM((KH,), jnp.float32),
            pltpu.VMEM((2 * KH,), jnp.float32),
            pltpu.VMEM((NGROUP, GATHER_GROUP), jnp.int32),
            pltpu.VMEM((ELEMS_PER_TILE,), jnp.float32),
            pltpu.VMEM((128,), jnp.float32),
            pltpu.SemaphoreType.DMA,
            pltpu.SemaphoreType.DMA,
            pltpu.SemaphoreType.DMA,
        ],
    )(_sc_body)
    part = k(table, ind_pad, buf)
    num = part[:, LANES:2 * LANES].sum()
    return part[:, :LANES].sum() / (num + 0.0001)


def kernel(output, mask, ind, target):
    table = jnp.transpose(
        output.reshape(B, C, 512 // 8, 8, 512 // 128, 128),
        (0, 1, 2, 4, 3, 5)).reshape(B * C * HW)
    ind_pad = jnp.pad(ind.astype(jnp.int32), ((0, 0), (0, KPAD - K)))
    maskf = jnp.pad(mask.astype(jnp.float32), ((0, 0), (0, KPAD - K)))
    t4 = jnp.transpose(
        jnp.pad(target, ((0, 0), (0, KPAD - K), (0, 0))).reshape(
            B, KPAD // 128, 128, C),
        (0, 1, 3, 2)).reshape(B, 2 * KPAD)
    buf = jnp.concatenate([maskf, t4], axis=1)
    return _reg_loss_sc(table, ind_pad, buf)
